# Initial kernel scaffold; baseline (speedup 1.0000x reference)
import jax
import jax.numpy as jnp
from jax.experimental import pallas as pl


def kernel(embedding_input, h_input, edge_index, W_fc, W_attn):
    raise NotImplementedError("write your pallas kernel here")


# speedup vs baseline: 12.1359x; 12.1359x over previous
import functools

import jax
import jax.numpy as jnp
from jax import lax
from jax.experimental import pallas as pl
from jax.experimental.pallas import tpu as pltpu
from jax.experimental.pallas import tpu_sc as plsc

N = 10000
D = 128
E = 320000
ALPHA = 0.7
BETA = 0.3
GAMMA = 0.2

NC = 2
NS = 16
NW = NC * NS
EPT = E // NW
ROWS = 80
RPAD = ROWS * 128

_mesh = plsc.VectorSubcoreMesh(core_axis_name="c", subcore_axis_name="s")


def _scores_body(wa_ref, wfc_ref, h_ref, out_ref):
    A = jnp.dot(wa_ref[...], wfc_ref[...], preferred_element_type=jnp.float32)
    s = lax.dot_general(
        A, h_ref[...], (((1,), (1,)), ((), ())),
        preferred_element_type=jnp.float32)
    out_ref[...] = jnp.pad(s, ((0, 0), (0, RPAD - N)))


def _scores(wa, wfc, h):
    return pl.pallas_call(
        _scores_body,
        out_shape=jax.ShapeDtypeStruct((2, RPAD), jnp.float32),
    )(wa, wfc, h)


def _edge_body(s_hbm, src_hbm, dst_hbm, p_hbm, dpart_hbm,
               s1_t, s2_t, src_t, dst_t, p_t, zb_t, dsum, sem):
    cid = lax.axis_index("c")
    sid = lax.axis_index("s")
    wid = sid * NC + cid

    pltpu.sync_copy(s_hbm.at[0], s1_t)
    pltpu.sync_copy(s_hbm.at[1], s2_t)
    pltpu.sync_copy(src_hbm.at[wid], src_t)
    pltpu.sync_copy(dst_hbm.at[wid], dst_t)

    for c in range(128):
        zb_t[pl.ds(c * 16, 16)] = jnp.zeros((16,), jnp.float32)

    @pl.when(sid == 0)
    def _():
        for k in range(RPAD // 2048):
            pltpu.sync_copy(zb_t, dsum.at[pl.ds(k * 2048, 2048)])

    plsc.subcore_barrier()

    iota16 = lax.broadcasted_iota(jnp.int32, (16,), 0)

    def row(r, _):
        def col(c, _):
            srcv = src_t[r, 0, pl.ds(c * 16, 16)]
            dstv = dst_t[r, 0, pl.ds(c * 16, 16)]
            s1v = plsc.load_gather(s1_t, [srcv])
            s2v = plsc.load_gather(s2_t, [dstv])
            a = s1v + s2v
            e = jnp.where(a > 0, a, GAMMA * a)
            e = jnp.minimum(e, 60.0)
            p = jnp.exp(e)
            lid = r * 128 + c * 16 + iota16
            p = jnp.where(lid < EPT, p, 0.0)
            p_t[r, 0, pl.ds(c * 16, 16)] = p
            return 0
        lax.fori_loop(0, 8, col, 0)
        return 0
    lax.fori_loop(0, ROWS, row, 0)

    def srow(r, _):
        pltpu.sync_copy(p_t.at[r, 0], dsum.at[dst_t.at[r, 0]], add=True)
        return 0
    lax.fori_loop(0, ROWS, srow, 0)

    plsc.subcore_barrier()

    pltpu.sync_copy(p_t, p_hbm.at[wid])

    @pl.when(sid == 0)
    def _():
        pltpu.sync_copy(dsum, dpart_hbm.at[cid])


@functools.partial(
    pl.kernel,
    out_type=(jax.ShapeDtypeStruct((NW, ROWS, 1, 128), jnp.float32),
              jax.ShapeDtypeStruct((NC, RPAD), jnp.float32)),
    mesh=_mesh,
    compiler_params=pltpu.CompilerParams(needs_layout_passes=False),
    scratch_types=[
        pltpu.VMEM((RPAD,), jnp.float32),
        pltpu.VMEM((RPAD,), jnp.float32),
        pltpu.VMEM((ROWS, 1, 128), jnp.int32),
        pltpu.VMEM((ROWS, 1, 128), jnp.int32),
        pltpu.VMEM((ROWS, 1, 128), jnp.float32),
        pltpu.VMEM((2048,), jnp.float32),
        pltpu.VMEM_SHARED((RPAD,), jnp.float32),
        pltpu.SemaphoreType.DMA,
    ],
)
def _edge_kernel(s_hbm, src_hbm, dst_hbm, p_hbm, dpart_hbm, *scratch):
    _edge_body(s_hbm, src_hbm, dst_hbm, p_hbm, dpart_hbm, *scratch)


def _msg_body(dpart_hbm, p_hbm, src_hbm, dst_hbm, h_hbm, acc_hbm,
              denom_t, dtmp_t, p_t, srcr, dstr, rowb, acc, sem):
    cid = lax.axis_index("c")
    sid = lax.axis_index("s")
    wid = sid * NC + cid

    pltpu.sync_copy(dpart_hbm.at[0], denom_t)
    pltpu.sync_copy(dpart_hbm.at[1], dtmp_t)

    def dadd(i, _):
        denom_t[pl.ds(i * 16, 16)] = (denom_t[pl.ds(i * 16, 16)]
                                      + dtmp_t[pl.ds(i * 16, 16)])
        return 0
    lax.fori_loop(0, RPAD // 16, dadd, 0)

    pltpu.sync_copy(p_hbm.at[wid], p_t)

    def arow(r, _):
        pltpu.sync_copy(dst_hbm.at[wid, r], dstr)

        def acol(c, _):
            dstv = dstr[0, pl.ds(c * 16, 16)]
            dv = plsc.load_gather(denom_t, [dstv])
            pv = p_t[r, 0, pl.ds(c * 16, 16)]
            p_t[r, 0, pl.ds(c * 16, 16)] = pv / (dv + 1e-16)
            return 0
        lax.fori_loop(0, 8, acol, 0)
        return 0
    lax.fori_loop(0, ROWS, arow, 0)

    def zrow(i, _):
        def zcol(c, _):
            rowb[i, pl.ds(c * 16, 16)] = jnp.zeros((16,), jnp.float32)
            return 0
        lax.fori_loop(0, 8, zcol, 0)
        return 0
    lax.fori_loop(0, 128, zrow, 0)
    base = sid * 640
    for k in range(5):
        pltpu.sync_copy(rowb, acc.at[pl.ds(base + k * 128, 128)])

    plsc.subcore_barrier()

    dnums = lax.GatherDimensionNumbers(
        offset_dims=(), collapsed_slice_dims=(0,), start_index_map=(0,))

    def mrow(r, _):
        pltpu.sync_copy(src_hbm.at[wid, r], srcr)
        pltpu.sync_copy(dst_hbm.at[wid, r], dstr)
        pltpu.async_copy(h_hbm.at[srcr.at[0]], rowb, sem).wait()

        def mcol(c, _):
            attnv = p_t[r, 0, pl.ds(c * 16, 16)]
            for j in range(16):
                sp = lax.gather(attnv, jnp.full((16, 1), j, jnp.int32),
                                dnums, (1,),
                                mode=lax.GatherScatterMode.PROMISE_IN_BOUNDS)
                row = c * 16 + j
                for dch in range(8):
                    rowb[row, pl.ds(dch * 16, 16)] = (
                        rowb[row, pl.ds(dch * 16, 16)] * sp)
            return 0
        lax.fori_loop(0, 8, mcol, 0)

        pltpu.sync_copy(rowb, acc.at[dstr.at[0]], add=True)
        return 0
    lax.fori_loop(0, ROWS, mrow, 0)

    plsc.subcore_barrier()

    for k in range(5):
        pltpu.sync_copy(acc.at[pl.ds(base + k * 128, 128)],
                        acc_hbm.at[cid, pl.ds(base + k * 128, 128)])


@functools.partial(
    pl.kernel,
    out_type=jax.ShapeDtypeStruct((NC, RPAD, D), jnp.float32),
    mesh=_mesh,
    compiler_params=pltpu.CompilerParams(needs_layout_passes=False),
    scratch_types=[
        pltpu.VMEM((RPAD,), jnp.float32),
        pltpu.VMEM((RPAD,), jnp.float32),
        pltpu.VMEM((ROWS, 1, 128), jnp.float32),
        pltpu.VMEM((1, 128), jnp.int32),
        pltpu.VMEM((1, 128), jnp.int32),
        pltpu.VMEM((128, D), jnp.float32),
        pltpu.VMEM_SHARED((RPAD, D), jnp.float32),
        pltpu.SemaphoreType.DMA,
    ],
)
def _msg_kernel(dpart_hbm, p_hbm, src_hbm, dst_hbm, h_hbm, acc_hbm, *scratch):
    _msg_body(dpart_hbm, p_hbm, src_hbm, dst_hbm, h_hbm, acc_hbm, *scratch)


def _blend_body(emb_ref, acc_ref, out_ref):
    out_ref[...] = (ALPHA * emb_ref[...]
                    + BETA * (acc_ref[0] + acc_ref[1])) / (ALPHA + BETA)


def _blend(emb, acc):
    blk = 1000
    return pl.pallas_call(
        _blend_body,
        grid=(N // blk,),
        in_specs=[pl.BlockSpec((blk, D), lambda g: (g, 0)),
                  pl.BlockSpec((NC, blk, D), lambda g: (0, g, 0))],
        out_specs=pl.BlockSpec((blk, D), lambda g: (g, 0)),
        out_shape=jax.ShapeDtypeStruct((N, D), jnp.float32),
    )(emb, acc)


def kernel(embedding_input, h_input, edge_index, W_fc, W_attn):
    wa = W_attn.reshape(2, D)
    s = _scores(wa, W_fc, h_input)

    src = edge_index[0].reshape(NW, EPT)
    dst = edge_index[1].reshape(NW, EPT)
    src = jnp.pad(src, ((0, 0), (0, RPAD - EPT))).reshape(NW, ROWS, 1, 128)
    dst = jnp.pad(dst, ((0, 0), (0, RPAD - EPT))).reshape(NW, ROWS, 1, 128)

    p, dpart = _edge_kernel(s, src, dst)
    acc = _msg_kernel(dpart, p, src, dst, h_input)
    return _blend(embedding_input, acc)


# speedup vs baseline: 16.8454x; 1.3881x over previous
import functools

import jax
import jax.numpy as jnp
from jax import lax
from jax.experimental import pallas as pl
from jax.experimental.pallas import tpu as pltpu
from jax.experimental.pallas import tpu_sc as plsc

N = 10000
D = 128
E = 320000
ALPHA = 0.7
BETA = 0.3
GAMMA = 0.2

NC = 2
NS = 16
NW = NC * NS
EPT = E // NW
ROWS = 80
RPAD = ROWS * 128

_mesh = plsc.VectorSubcoreMesh(core_axis_name="c", subcore_axis_name="s")


def _scores_body(wa_ref, wfc_ref, h_ref, out_ref):
    A = jnp.dot(wa_ref[...], wfc_ref[...], preferred_element_type=jnp.float32)
    s = lax.dot_general(
        A, h_ref[...], (((1,), (1,)), ((), ())),
        preferred_element_type=jnp.float32)
    out_ref[...] = jnp.pad(s, ((0, 0), (0, RPAD - N)))


def _scores(wa, wfc, h):
    return pl.pallas_call(
        _scores_body,
        out_shape=jax.ShapeDtypeStruct((2, RPAD), jnp.float32),
    )(wa, wfc, h)


def _edge_body(s_hbm, sd_hbm, p_hbm, dpart_hbm,
               s1_t, s2_t, sd_t, p_t, zb_t, dsum, sem):
    cid = lax.axis_index("c")
    sid = lax.axis_index("s")
    wid = sid * NC + cid

    pltpu.sync_copy(s_hbm.at[0], s1_t)
    pltpu.sync_copy(s_hbm.at[1], s2_t)
    pltpu.sync_copy(sd_hbm.at[wid], sd_t)

    for c in range(128):
        zb_t[pl.ds(c * 16, 16)] = jnp.zeros((16,), jnp.float32)

    @pl.when(sid == 0)
    def _():
        for k in range(RPAD // 2048):
            pltpu.sync_copy(zb_t, dsum.at[pl.ds(k * 2048, 2048)])

    plsc.subcore_barrier()

    iota16 = lax.broadcasted_iota(jnp.int32, (16,), 0)

    def row(r, _):
        def col(c, _):
            srcv = sd_t[r, 0, pl.ds(c * 16, 16)]
            dstv = sd_t[r, 1, pl.ds(c * 16, 16)]
            s1v = plsc.load_gather(s1_t, [srcv])
            s2v = plsc.load_gather(s2_t, [dstv])
            a = s1v + s2v
            e = jnp.where(a > 0, a, GAMMA * a)
            e = jnp.minimum(e, 60.0)
            p = jnp.exp(e)
            lid = r * 128 + c * 16 + iota16
            p = jnp.where(lid < EPT, p, 0.0)
            p_t[r, 0, pl.ds(c * 16, 16)] = p
            return 0
        lax.fori_loop(0, 8, col, 0)
        return 0
    lax.fori_loop(0, ROWS, row, 0)

    def srow(r, _):
        pltpu.sync_copy(p_t.at[r, 0], dsum.at[sd_t.at[r, 1]], add=True)
        return 0
    lax.fori_loop(0, ROWS, srow, 0)

    plsc.subcore_barrier()

    pltpu.sync_copy(p_t, p_hbm.at[wid])

    @pl.when(sid == 0)
    def _():
        pltpu.sync_copy(dsum, dpart_hbm.at[cid])


@functools.partial(
    pl.kernel,
    out_type=(jax.ShapeDtypeStruct((NW, ROWS, 1, 128), jnp.float32),
              jax.ShapeDtypeStruct((NC, RPAD), jnp.float32)),
    mesh=_mesh,
    compiler_params=pltpu.CompilerParams(needs_layout_passes=False),
    scratch_types=[
        pltpu.VMEM((RPAD,), jnp.float32),
        pltpu.VMEM((RPAD,), jnp.float32),
        pltpu.VMEM((ROWS, 2, 128), jnp.int32),
        pltpu.VMEM((ROWS, 1, 128), jnp.float32),
        pltpu.VMEM((2048,), jnp.float32),
        pltpu.VMEM_SHARED((RPAD,), jnp.float32),
        pltpu.SemaphoreType.DMA,
    ],
)
def _edge_kernel(s_hbm, sd_hbm, p_hbm, dpart_hbm, *scratch):
    _edge_body(s_hbm, sd_hbm, p_hbm, dpart_hbm, *scratch)


def _dmerge_body(d_ref, out_ref):
    out_ref[...] = d_ref[0] + d_ref[1]


def _dmerge(dpart):
    return pl.pallas_call(
        _dmerge_body,
        out_shape=jax.ShapeDtypeStruct((ROWS, 128), jnp.float32),
    )(dpart.reshape(NC, ROWS, 128))


_DNUMS = lax.GatherDimensionNumbers(
    offset_dims=(), collapsed_slice_dims=(0,), start_index_map=(0,))


def _msg_body(denom_hbm, p_hbm, sd_hbm, h_hbm, acc_hbm,
              denom_t, sd0, sd1, sd2, sd3, pr0, pr1, pr2, pr3,
              rba, rbb, acc,
              si0, si1, si2, si3, sga, sgb, ssa, ssb):
    cid = lax.axis_index("c")
    sid = lax.axis_index("s")
    wid = sid * NC + cid
    sd = (sd0, sd1, sd2, sd3)
    pr = (pr0, pr1, pr2, pr3)
    si = (si0, si1, si2, si3)
    rb = (rba, rbb)
    sg = (sga, sgb)
    ss = (ssa, ssb)

    pltpu.sync_copy(denom_hbm, denom_t)

    def zrow(i, _):
        def zcol(c, _):
            rba[i, pl.ds(c * 16, 16)] = jnp.zeros((16,), jnp.float32)
            return 0
        lax.fori_loop(0, 8, zcol, 0)
        return 0
    lax.fori_loop(0, 128, zrow, 0)
    base = sid * 640
    for k in range(5):
        pltpu.sync_copy(rba, acc.at[pl.ds(base + k * 128, 128)])

    plsc.subcore_barrier()

    def start_idx(r, s):
        pltpu.async_copy(sd_hbm.at[wid, r], sd[s], si[s])
        pltpu.async_copy(p_hbm.at[wid, r], pr[s], si[s])

    def wait_idx(r, s):
        pltpu.make_async_copy(sd_hbm.at[wid, r], sd[s], si[s]).wait()
        pltpu.make_async_copy(p_hbm.at[wid, r], pr[s], si[s]).wait()

    def start_gather(s, b):
        pltpu.async_copy(h_hbm.at[sd[s].at[0]], rb[b], sg[b])

    def wait_gather(b):
        pltpu.make_async_copy(h_hbm.at[sd[0].at[0]], rb[b], sg[b]).wait()

    def start_scatter(s, b):
        pltpu.async_copy(rb[b], acc.at[sd[s].at[1]], ss[b], add=True)

    def wait_scatter(b):
        pltpu.make_async_copy(rb[b], acc.at[sd[0].at[1]], ss[b]).wait()

    start_idx(0, 0)
    start_idx(1, 1)
    wait_idx(0, 0)
    start_gather(0, 0)

    def quad(i, _):
        for k in range(4):
            r = i * 4 + k
            b = k % 2
            s = k

            @pl.when(r + 2 < ROWS)
            def _():
                start_idx(r + 2, (k + 2) % 4)

            @pl.when(r + 1 < ROWS)
            def _():
                wait_idx(r + 1, (k + 1) % 4)

                @pl.when(r >= 1)
                def _():
                    wait_scatter(1 - b)
                start_gather((k + 1) % 4, 1 - b)

            wait_gather(b)

            def scale(c, _):
                dstv = sd[s][1, pl.ds(c * 16, 16)]
                dv = plsc.load_gather(denom_t, [dstv])
                pv = pr[s][0, pl.ds(c * 16, 16)]
                attnv = pv / (dv + 1e-16)
                for j in range(16):
                    sp = lax.gather(
                        attnv, jnp.full((16, 1), j, jnp.int32), _DNUMS, (1,),
                        mode=lax.GatherScatterMode.PROMISE_IN_BOUNDS)
                    row = c * 16 + j
                    for dch in range(8):
                        rb[b][row, pl.ds(dch * 16, 16)] = (
                            rb[b][row, pl.ds(dch * 16, 16)] * sp)
                return 0
            lax.fori_loop(0, 8, scale, 0)

            start_scatter(s, b)
        return 0
    lax.fori_loop(0, ROWS // 4, quad, 0)

    wait_scatter(0)
    wait_scatter(1)

    plsc.subcore_barrier()

    for k in range(5):
        pltpu.sync_copy(acc.at[pl.ds(base + k * 128, 128)],
                        acc_hbm.at[cid, pl.ds(base + k * 128, 128)])


@functools.partial(
    pl.kernel,
    out_type=jax.ShapeDtypeStruct((NC, RPAD, D), jnp.float32),
    mesh=_mesh,
    compiler_params=pltpu.CompilerParams(needs_layout_passes=False),
    scratch_types=[
        pltpu.VMEM((RPAD,), jnp.float32),
        pltpu.VMEM((2, 128), jnp.int32),
        pltpu.VMEM((2, 128), jnp.int32),
        pltpu.VMEM((2, 128), jnp.int32),
        pltpu.VMEM((2, 128), jnp.int32),
        pltpu.VMEM((1, 128), jnp.float32),
        pltpu.VMEM((1, 128), jnp.float32),
        pltpu.VMEM((1, 128), jnp.float32),
        pltpu.VMEM((1, 128), jnp.float32),
        pltpu.VMEM((128, D), jnp.float32),
        pltpu.VMEM((128, D), jnp.float32),
        pltpu.VMEM_SHARED((RPAD, D), jnp.float32),
        pltpu.SemaphoreType.DMA,
        pltpu.SemaphoreType.DMA,
        pltpu.SemaphoreType.DMA,
        pltpu.SemaphoreType.DMA,
        pltpu.SemaphoreType.DMA,
        pltpu.SemaphoreType.DMA,
        pltpu.SemaphoreType.DMA,
        pltpu.SemaphoreType.DMA,
    ],
)
def _msg_kernel(denom_hbm, p_hbm, sd_hbm, h_hbm, acc_hbm, *scratch):
    _msg_body(denom_hbm, p_hbm, sd_hbm, h_hbm, acc_hbm, *scratch)


def _blend_body(emb_ref, acc_ref, out_ref):
    out_ref[...] = (ALPHA * emb_ref[...]
                    + BETA * (acc_ref[0] + acc_ref[1])) / (ALPHA + BETA)


def _blend(emb, acc):
    blk = 1000
    return pl.pallas_call(
        _blend_body,
        grid=(N // blk,),
        in_specs=[pl.BlockSpec((blk, D), lambda g: (g, 0)),
                  pl.BlockSpec((NC, blk, D), lambda g: (0, g, 0))],
        out_specs=pl.BlockSpec((blk, D), lambda g: (g, 0)),
        out_shape=jax.ShapeDtypeStruct((N, D), jnp.float32),
    )(emb, acc)


def kernel(embedding_input, h_input, edge_index, W_fc, W_attn):
    wa = W_attn.reshape(2, D)
    s = _scores(wa, W_fc, h_input)

    src = edge_index[0].reshape(NW, EPT)
    dst = edge_index[1].reshape(NW, EPT)
    src = jnp.pad(src, ((0, 0), (0, RPAD - EPT))).reshape(NW, ROWS, 1, 128)
    dst = jnp.pad(dst, ((0, 0), (0, RPAD - EPT))).reshape(NW, ROWS, 1, 128)
    sd = jnp.concatenate([src, dst], axis=2)

    p, dpart = _edge_kernel(s, sd)
    denom = _dmerge(dpart).reshape(RPAD)
    acc = _msg_kernel(denom, p, sd, h_input)
    return _blend(embedding_input, acc)
